# Initial kernel scaffold; baseline (speedup 1.0000x reference)
import functools
import jax
import jax.numpy as jnp
from jax import lax
from jax.experimental import pallas as pl
from jax.experimental.pallas import tpu as pltpu, tpu_sc as plsc

BATCH = 16
NUM_POINTS = 8192
TABLE_ROWS = 1024
HALF = 32

NPAIRS = BATCH * NUM_POINTS
NROWS = 2 * NPAIRS
NW = 32
ROWS_PER_W = NROWS // NW
CHUNK = 1024
NCHUNK = ROWS_PER_W // CHUNK
GSIZE = 128
NG = CHUNK // GSIZE

_mesh = plsc.VectorSubcoreMesh(core_axis_name="c", subcore_axis_name="s")


@functools.partial(
    pl.kernel,
    out_type=jax.ShapeDtypeStruct((NROWS, HALF), jnp.float32),
    mesh=_mesh,
    scratch_types=[
        pltpu.VMEM((NG, GSIZE), jnp.int32),
        pltpu.VMEM((CHUNK, HALF), jnp.float32),
        pltpu.SemaphoreType.DMA,
    ],
)
def _sc_gather(coords_hbm, table_hbm, out_hbm, idx_v, rows_v, sem):
    wid = lax.axis_index("s") * 2 + lax.axis_index("c")
    offs = (lax.iota(jnp.int32, 16) & 1) * TABLE_ROWS

    def chunk_body(g, _):
        row0 = wid * ROWS_PER_W + g * CHUNK
        pltpu.sync_copy(coords_hbm.at[pl.ds(row0 // GSIZE, NG), :], idx_v)
        for j in range(NG):
            row = idx_v.at[j]

            def add_off(i, _):
                sl = pl.ds(i * 16, 16)
                row[sl] = row[sl] + offs
                return 0

            lax.fori_loop(0, GSIZE // 16, add_off, 0)
        copies = [
            pltpu.async_copy(
                table_hbm.at[idx_v.at[j]],
                rows_v.at[pl.ds(j * GSIZE, GSIZE), :],
                sem,
            )
            for j in range(NG)
        ]
        for c in copies:
            c.wait()
        pltpu.sync_copy(rows_v, out_hbm.at[pl.ds(row0, CHUNK), :])
        return 0

    lax.fori_loop(0, NCHUNK, chunk_body, 0)


def kernel(pixel_coordinates, x_table, y_table):
    coords = pixel_coordinates.reshape(NROWS // GSIZE, GSIZE)
    table = jnp.concatenate([x_table, y_table], axis=0)
    out = _sc_gather(coords, table)
    return out.reshape(BATCH, NUM_POINTS, 2 * HALF)


# speedup vs baseline: 4.2771x; 4.2771x over previous
import functools
import jax
import jax.numpy as jnp
from jax import lax
from jax.experimental import pallas as pl
from jax.experimental.pallas import tpu as pltpu, tpu_sc as plsc

BATCH = 16
NUM_POINTS = 8192
TABLE_ROWS = 1024
HALF = 32

NPAIRS = BATCH * NUM_POINTS
NROWS = 2 * NPAIRS
NW = 32
ROWS_PER_W = NROWS // NW
CHUNK = 1024
NCHUNK = ROWS_PER_W // CHUNK
GSIZE = 128
NG = CHUNK // GSIZE

_mesh = plsc.VectorSubcoreMesh(core_axis_name="c", subcore_axis_name="s")


@functools.partial(
    pl.kernel,
    out_type=jax.ShapeDtypeStruct((NROWS, HALF), jnp.float32),
    mesh=_mesh,
    scratch_types=[
        pltpu.VMEM((NG, GSIZE), jnp.int32),
        pltpu.VMEM((CHUNK, HALF), jnp.float32),
        pltpu.SemaphoreType.DMA,
    ],
    compiler_params=pltpu.CompilerParams(use_tc_tiling_on_sc=False),
)
def _sc_gather(coords_hbm, table_hbm, out_hbm, idx_v, rows_v, sem):
    wid = lax.axis_index("s") * 2 + lax.axis_index("c")
    offs = (lax.iota(jnp.int32, 16) & 1) * TABLE_ROWS

    def chunk_body(g, _):
        row0 = wid * ROWS_PER_W + g * CHUNK
        crow0 = pl.multiple_of(row0 // GSIZE, 8)
        pltpu.sync_copy(coords_hbm.at[pl.ds(crow0, NG), :], idx_v)
        for j in range(NG):
            row = idx_v.at[j]

            def add_off(i, _):
                sl = pl.ds(i * 16, 16)
                row[sl] = row[sl] + offs
                return 0

            lax.fori_loop(0, GSIZE // 16, add_off, 0)
        copies = [
            pltpu.async_copy(
                table_hbm.at[idx_v.at[j]],
                rows_v.at[pl.ds(j * GSIZE, GSIZE), :],
                sem,
            )
            for j in range(NG)
        ]
        for c in copies:
            c.wait()
        pltpu.sync_copy(rows_v, out_hbm.at[pl.ds(row0, CHUNK), :])
        return 0

    lax.fori_loop(0, NCHUNK, chunk_body, 0)


def kernel(pixel_coordinates, x_table, y_table):
    coords = pixel_coordinates.reshape(NROWS // GSIZE, GSIZE)
    table = jnp.concatenate([x_table, y_table], axis=0)
    out = _sc_gather(coords, table)
    return out.reshape(BATCH, NUM_POINTS, 2 * HALF)
